# Initial kernel scaffold; baseline (speedup 1.0000x reference)
import jax
import jax.numpy as jnp
from jax.experimental import pallas as pl


def kernel(x, target_id, embed, embed_out):
    raise NotImplementedError("write your pallas kernel here")


# speedup vs baseline: 1.0514x; 1.0514x over previous
import functools

import jax
import jax.numpy as jnp
from jax import lax
from jax.experimental import pallas as pl
from jax.experimental.pallas import tpu as pltpu
from jax.experimental.pallas import tpu_sc as plsc

NDIM = 32
CTX = 20
NW = 32
GROW = 128


def _sc_kernel(batch):
    bpw = batch // NW
    sc_chunks = 8
    cb = bpw // sc_chunks
    rows = cb * CTX
    gi = rows // GROW
    idx_rows = bpw * CTX // GROW
    tid_rows = bpw // GROW

    mesh = plsc.VectorSubcoreMesh(core_axis_name="c", subcore_axis_name="s")

    @functools.partial(
        pl.kernel,
        mesh=mesh,
        out_type=jax.ShapeDtypeStruct((batch,), jnp.float32),
        compiler_params=pltpu.CompilerParams(
            needs_layout_passes=False, use_tc_tiling_on_sc=False),
        scratch_types=[
            pltpu.VMEM((idx_rows, GROW), jnp.int32),
            pltpu.VMEM((tid_rows, GROW), jnp.int32),
            pltpu.VMEM((rows, NDIM), jnp.float32),
            pltpu.VMEM((rows, NDIM), jnp.float32),
            pltpu.VMEM((bpw, NDIM), jnp.float32),
            pltpu.VMEM((cb, 16), jnp.float32),
            pltpu.VMEM((bpw,), jnp.float32),
            pltpu.SemaphoreType.DMA,
            pltpu.SemaphoreType.DMA,
            pltpu.SemaphoreType.DMA,
        ],
    )
    def body(x2d, tid2d, embed, embed_out, out, idx_v, tid_v, buf_a, buf_b,
             tgt_v, prod_v, out_v, sem_a, sem_b, sem_t):
        wid = lax.axis_index("s") * 2 + lax.axis_index("c")
        base = wid * bpw

        pltpu.sync_copy(x2d.at[pl.ds(wid * idx_rows, idx_rows)], idx_v)
        pltpu.sync_copy(tid2d.at[pl.ds(wid * tid_rows, tid_rows)], tid_v)

        tgt_dmas = []
        for g in range(tid_rows):
            tgt_dmas.append(pltpu.async_copy(
                embed_out.at[tid_v.at[g]],
                tgt_v.at[pl.ds(g * GROW, GROW)], sem_t))
        for d in tgt_dmas:
            d.wait()

        bufs = (buf_a, buf_b)
        sems = (sem_a, sem_b)

        def fire(s):
            dmas = []
            buf = bufs[s % 2]
            sem = sems[s % 2]
            for g in range(gi):
                dmas.append(pltpu.async_copy(
                    embed.at[idx_v.at[s * gi + g]],
                    buf.at[pl.ds(g * GROW, GROW)], sem))
            return dmas

        inflight = fire(0)
        lanes = lax.iota(jnp.int32, 16)

        for s in range(sc_chunks):
            nxt = fire(s + 1) if s + 1 < sc_chunks else []
            for d in inflight:
                d.wait()
            inflight = nxt
            buf = bufs[s % 2]

            def pool(c, _):
                e = s * cb + c
                r0 = c * CTX
                acc0 = buf[r0, pl.ds(0, 16)]
                acc1 = buf[r0, pl.ds(16, 16)]
                for j in range(1, CTX):
                    acc0 = acc0 + buf[r0 + j, pl.ds(0, 16)]
                    acc1 = acc1 + buf[r0 + j, pl.ds(16, 16)]
                t0 = tgt_v[e, pl.ds(0, 16)]
                t1 = tgt_v[e, pl.ds(16, 16)]
                prod_v[c, pl.ds(0, 16)] = acc0 * t0 + acc1 * t1
                return 0
            lax.fori_loop(0, cb, pool, 0, unroll=False)

            for g in range(cb // 16):
                cvec = g * 16 + lanes
                acc = plsc.load_gather(prod_v, [cvec, lanes * 0])
                for l in range(1, 16):
                    acc = acc + plsc.load_gather(prod_v, [cvec, lanes * 0 + l])
                out_v[pl.ds(s * cb + g * 16, 16)] = acc

        pltpu.sync_copy(out_v, out.at[pl.ds(base, bpw)])

    return body


def kernel(x, target_id, embed, embed_out):
    batch, ctx = x.shape
    assert ctx == CTX
    x2d = x.reshape(batch * CTX // GROW, GROW)
    tid2d = target_id.reshape(batch // GROW, GROW)
    return _sc_kernel(batch)(x2d, tid2d, embed, embed_out)
